# baseline (speedup 1.0000x reference)
import jax
import jax.numpy as jnp
from jax.experimental import pallas as pl
from jax.experimental.pallas import tpu as pltpu

_EPS = 1e-8


def _read_head_body(x_ref, wp_ref, W_ref, b_ref, mem_ref, r_ref, w_ref):
    BB = x_ref.shape[0]
    N, M = mem_ref.shape[1], mem_ref.shape[2]

    o = jax.lax.dot_general(
        x_ref[...], W_ref[...], (((1,), (1,)), ((), ())),
        preferred_element_type=jnp.float32) + b_ref[...]
    k = o[:, :M]
    beta = jax.nn.softplus(o[:, M:M + 1])
    g = jax.nn.sigmoid(o[:, M + 1:M + 2])
    s = jax.nn.softmax(o[:, M + 2:M + 5], axis=1)
    gamma = 1.0 + jax.nn.softplus(o[:, M + 5:M + 6])

    mem = mem_ref[...]
    dot = jnp.sum(mem * k[:, None, :], axis=2)
    nsq = jnp.sum(mem * mem, axis=2)
    knorm = jnp.sqrt(jnp.sum(k * k, axis=1, keepdims=True))
    denom = jnp.sqrt(nsq) * knorm + _EPS
    sim = beta * (dot / denom)

    sim = sim - jnp.max(sim, axis=1, keepdims=True)
    e = jnp.exp(sim)
    w_c = e / jnp.sum(e, axis=1, keepdims=True)

    w_g = g * w_c + (1.0 - g) * wp_ref[...]

    w_m = jnp.concatenate([w_g[:, -1:], w_g[:, :-1]], axis=1)
    w_p = jnp.concatenate([w_g[:, 1:], w_g[:, :1]], axis=1)
    w_t = s[:, 0:1] * w_m + s[:, 1:2] * w_g + s[:, 2:3] * w_p

    w_pow = w_t ** gamma
    wn = w_pow / (jnp.sum(w_pow, axis=1, keepdims=True) + 1e-16)
    w_ref[...] = wn

    r_ref[...] = jnp.sum(mem * wn[:, :, None], axis=1)


def _read_head(x, w_prev, mem, W, b, *, bb=8, interpret=False):
    B, C = x.shape
    _, N, M = mem.shape
    b2 = b.reshape(1, M + 6)
    grid = (B // bb,)
    return pl.pallas_call(
        _read_head_body,
        out_shape=(
            jax.ShapeDtypeStruct((B, M), jnp.float32),
            jax.ShapeDtypeStruct((B, N), jnp.float32),
        ),
        grid=grid,
        in_specs=[
            pl.BlockSpec((bb, C), lambda i: (i, 0)),
            pl.BlockSpec((bb, N), lambda i: (i, 0)),
            pl.BlockSpec((M + 6, C), lambda i: (0, 0)),
            pl.BlockSpec((1, M + 6), lambda i: (0, 0)),
            pl.BlockSpec((bb, N, M), lambda i: (i, 0, 0)),
        ],
        out_specs=(
            pl.BlockSpec((bb, M), lambda i: (i, 0)),
            pl.BlockSpec((bb, N), lambda i: (i, 0)),
        ),
        compiler_params=pltpu.CompilerParams(
            dimension_semantics=("parallel",),
            vmem_limit_bytes=56 * 1024 * 1024,
        ),
        name="ntm_read_head",
        interpret=interpret,
    )(x, w_prev, W, b2, mem)


def kernel(x, w_prev, mem, W, b):
    r, w = _read_head(x, w_prev, mem, W, b)
    return (r, w)


# speedup vs baseline: 2.4265x; 2.4265x over previous
import jax
import jax.numpy as jnp
from jax.experimental import pallas as pl
from jax.experimental.pallas import tpu as pltpu

_EPS = 1e-8


def _read_head_body(x_ref, wp_ref, W_ref, b_ref, mem_ref, r_ref, w_ref,
                    dn_ref, mp_ref, wnp_ref):
    BB = x_ref.shape[0]
    N, M = mem_ref.shape[1], mem_ref.shape[2]
    R = BB * N // 128

    lane_b = jax.lax.broadcasted_iota(jnp.int32, (BB, BB * N), 1) // N
    row_b = jax.lax.broadcasted_iota(jnp.int32, (BB, BB * N), 0)
    wnp_bd = jnp.where(lane_b == row_b, jnp.tile(wnp_ref[...], (1, BB)), 0.0)
    r_ref[...] = jax.lax.dot_general(
        wnp_bd, mp_ref[...].reshape(BB * N, M), (((1,), (0,)), ((), ())),
        preferred_element_type=jnp.float32)

    o = jax.lax.dot_general(
        x_ref[...], W_ref[...], (((1,), (1,)), ((), ())),
        preferred_element_type=jnp.float32) + b_ref[...]
    k = o[:, :M]
    beta = jax.nn.softplus(o[:, M:M + 1])
    g = jax.nn.sigmoid(o[:, M + 1:M + 2])
    s = jax.nn.softmax(o[:, M + 2:M + 5], axis=1)
    gamma = 1.0 + jax.nn.softplus(o[:, M + 5:M + 6])
    knorm = jnp.sqrt(jnp.sum(k * k, axis=1, keepdims=True))

    mem = mem_ref[...]
    pk = mem * k[:, None, :]
    sq = mem * mem
    cat = jnp.concatenate([pk, sq], axis=2)
    flat = cat.reshape(BB * N, 2 * M)

    rid = jax.lax.broadcasted_iota(jnp.int32, (2 * M, 2), 0)
    cid = jax.lax.broadcasted_iota(jnp.int32, (2 * M, 2), 1)
    sel = jnp.where((rid < M) == (cid == 0), 1.0, 0.0).astype(jnp.float32)

    half = BB * N // 2
    dn_t = jax.lax.dot_general(flat[:half], sel, (((1,), (0,)), ((), ())),
                               preferred_element_type=jnp.float32)
    dn_b = jax.lax.dot_general(flat[half:], sel, (((1,), (0,)), ((), ())),
                               preferred_element_type=jnp.float32)
    dn3_t = jnp.swapaxes(dn_t.reshape(R // 2, 128, 2), 1, 2)
    dn3_b = jnp.swapaxes(dn_b.reshape(R // 2, 128, 2), 1, 2)
    dn_ref[0] = jnp.concatenate([dn3_t[:, 0, :], dn3_b[:, 0, :]], axis=0)
    dn_ref[1] = jnp.concatenate([dn3_t[:, 1, :], dn3_b[:, 1, :]], axis=0)
    dot = dn_ref[0]
    nsq = dn_ref[1]

    def bscal(v):
        return jnp.broadcast_to(v[:, :, None], (BB, N // 128, 128)).reshape(R, 128)

    denom = jnp.sqrt(nsq) * bscal(knorm) + _EPS
    sim = bscal(beta) * (dot / denom)

    sim3 = sim.reshape(BB, N // 128, 128)
    e = jnp.exp(sim3)
    w_c = e / jnp.sum(e, axis=(1, 2), keepdims=True)

    g3 = g[:, :, None]
    w_g = g3 * w_c + (1.0 - g3) * wp_ref[...]

    last = w_g[:, :, -1:]
    first = w_g[:, :, :1]
    prev_last = jnp.concatenate([last[:, -1:], last[:, :-1]], axis=1)
    next_first = jnp.concatenate([first[:, 1:], first[:, :1]], axis=1)
    w_m = jnp.concatenate([prev_last, w_g[:, :, :-1]], axis=2)
    w_p = jnp.concatenate([w_g[:, :, 1:], next_first], axis=2)
    s0 = s[:, 0][:, None, None]
    s1 = s[:, 1][:, None, None]
    s2 = s[:, 2][:, None, None]
    w_t = s0 * w_m + s1 * w_g + s2 * w_p

    w_pow = w_t ** gamma[:, :, None]
    wn = w_pow / (jnp.sum(w_pow, axis=(1, 2), keepdims=True) + 1e-16)

    wn2 = jnp.concatenate([wn[:, h, :] for h in range(N // 128)], axis=1)
    w_ref[...] = wn2

    wnp_ref[...] = wn2
    mp_ref[...] = mem


def _read_head(x, w_prev, mem, W, b, *, bb=8, interpret=False):
    B, C = x.shape
    _, N, M = mem.shape
    b2 = b.reshape(1, M + 6)
    wp3 = w_prev.reshape(B, N // 128, 128)
    NB = B // bb
    last = NB - 1
    grid = (NB + 1,)
    r, w3 = pl.pallas_call(
        _read_head_body,
        out_shape=(
            jax.ShapeDtypeStruct((B, M), jnp.float32),
            jax.ShapeDtypeStruct((B, N), jnp.float32),
        ),
        grid=grid,
        in_specs=[
            pl.BlockSpec((bb, C), lambda i: (jnp.minimum(i, last), 0)),
            pl.BlockSpec((bb, N // 128, 128),
                         lambda i: (jnp.minimum(i, last), 0, 0)),
            pl.BlockSpec((M + 6, C), lambda i: (0, 0)),
            pl.BlockSpec((1, M + 6), lambda i: (0, 0)),
            pl.BlockSpec((bb, N, M), lambda i: (jnp.minimum(i, last), 0, 0)),
        ],
        out_specs=(
            pl.BlockSpec((bb, M), lambda i: (jnp.maximum(i - 1, 0), 0)),
            pl.BlockSpec((bb, N), lambda i: (jnp.minimum(i, last), 0)),
        ),
        scratch_shapes=[pltpu.VMEM((2, N * bb // 128, 128), jnp.float32),
                        pltpu.VMEM((bb, N, M), jnp.float32),
                        pltpu.VMEM((bb, N), jnp.float32)],
        compiler_params=pltpu.CompilerParams(
            dimension_semantics=("parallel",),
            vmem_limit_bytes=56 * 1024 * 1024,
        ),
        name="ntm_read_head",
        interpret=interpret,
    )(x, wp3, W, b2, mem)
    return r, w3


def kernel(x, w_prev, mem, W, b):
    r, w = _read_head(x, w_prev, mem, W, b)
    return (r, w)


# speedup vs baseline: 2.5790x; 1.0629x over previous
import jax
import jax.numpy as jnp
from jax.experimental import pallas as pl
from jax.experimental.pallas import tpu as pltpu

_EPS = 1e-8


def _read_head_body(x_ref, wp_ref, W_ref, b_ref, mem_ref, r_ref, w_ref, dn_ref):
    BB = x_ref.shape[0]
    N, M = mem_ref.shape[1], mem_ref.shape[2]

    o = jax.lax.dot_general(
        x_ref[...], W_ref[...], (((1,), (1,)), ((), ())),
        preferred_element_type=jnp.float32) + b_ref[...]
    k = o[:, :M]
    beta = jax.nn.softplus(o[:, M:M + 1])
    g = jax.nn.sigmoid(o[:, M + 1:M + 2])
    s = jax.nn.softmax(o[:, M + 2:M + 5], axis=1)
    gamma = 1.0 + jax.nn.softplus(o[:, M + 5:M + 6])
    knorm = jnp.sqrt(jnp.sum(k * k, axis=1, keepdims=True))

    mem = mem_ref[...]

    rid = jax.lax.broadcasted_iota(jnp.int32, (2 * M, 2), 0)
    cid = jax.lax.broadcasted_iota(jnp.int32, (2 * M, 2), 1)
    sel = jnp.where((rid < M) == (cid == 0), 1.0, 0.0).astype(jnp.float32)

    C = 4
    NC = N // C
    HC = NC // 128
    parts = []
    for c in range(C):
        mc = mem[:, c * NC:(c + 1) * NC, :]
        catc = jnp.concatenate([mc * k[:, None, :], mc * mc], axis=2)
        fc = catc.reshape(BB * NC, 2 * M)
        dnc = jax.lax.dot_general(fc, sel, (((1,), (0,)), ((), ())),
                                  preferred_element_type=jnp.float32)
        dn3 = jnp.swapaxes(dnc.reshape(BB * NC // 128, 128, 2), 1, 2)
        parts.append(dn3)
    dn_ref[0] = jnp.concatenate(
        [p.reshape(BB, HC, 2, 128)[:, :, 0, :] for p in parts], axis=1)
    dn_ref[1] = jnp.concatenate(
        [p.reshape(BB, HC, 2, 128)[:, :, 1, :] for p in parts], axis=1)
    dot = dn_ref[0]
    nsq = dn_ref[1]

    def bscal(v):
        return v[:, :, None]

    denom = jnp.sqrt(nsq) * bscal(knorm) + _EPS
    sim3 = bscal(beta) * (dot / denom)

    e = jnp.exp(sim3)
    w_c = e / jnp.sum(e, axis=(1, 2), keepdims=True)

    g3 = g[:, :, None]
    w_g = g3 * w_c + (1.0 - g3) * wp_ref[...]

    last = w_g[:, :, -1:]
    first = w_g[:, :, :1]
    prev_last = jnp.concatenate([last[:, -1:], last[:, :-1]], axis=1)
    next_first = jnp.concatenate([first[:, 1:], first[:, :1]], axis=1)
    w_m = jnp.concatenate([prev_last, w_g[:, :, :-1]], axis=2)
    w_p = jnp.concatenate([w_g[:, :, 1:], next_first], axis=2)
    s0 = s[:, 0][:, None, None]
    s1 = s[:, 1][:, None, None]
    s2 = s[:, 2][:, None, None]
    w_t = s0 * w_m + s1 * w_g + s2 * w_p

    w_pow = w_t ** gamma[:, :, None]
    wn = w_pow / (jnp.sum(w_pow, axis=(1, 2), keepdims=True) + 1e-16)

    wn2 = jnp.concatenate([wn[:, h, :] for h in range(N // 128)], axis=1)
    w_ref[...] = wn2

    lane_b = jax.lax.broadcasted_iota(jnp.int32, (BB, BB * N), 1) // N
    row_b = jax.lax.broadcasted_iota(jnp.int32, (BB, BB * N), 0)
    wn_bd = jnp.where(lane_b == row_b, jnp.tile(wn2, (1, BB)), 0.0)
    r_ref[...] = jax.lax.dot_general(
        wn_bd, mem.reshape(BB * N, M), (((1,), (0,)), ((), ())),
        preferred_element_type=jnp.float32)


def _read_head(x, w_prev, mem, W, b, *, bb=16, interpret=False):
    B, C = x.shape
    _, N, M = mem.shape
    b2 = b.reshape(1, M + 6)
    wp3 = w_prev.reshape(B, N // 128, 128)
    grid = (B // bb,)
    r, w3 = pl.pallas_call(
        _read_head_body,
        out_shape=(
            jax.ShapeDtypeStruct((B, M), jnp.float32),
            jax.ShapeDtypeStruct((B, N), jnp.float32),
        ),
        grid=grid,
        in_specs=[
            pl.BlockSpec((bb, C), lambda i: (i, 0)),
            pl.BlockSpec((bb, N // 128, 128), lambda i: (i, 0, 0)),
            pl.BlockSpec((M + 6, C), lambda i: (0, 0)),
            pl.BlockSpec((1, M + 6), lambda i: (0, 0)),
            pl.BlockSpec((bb, N, M), lambda i: (i, 0, 0)),
        ],
        out_specs=(
            pl.BlockSpec((bb, M), lambda i: (i, 0)),
            pl.BlockSpec((bb, N), lambda i: (i, 0)),
        ),
        scratch_shapes=[pltpu.VMEM((2, bb, N // 128, 128), jnp.float32)],
        compiler_params=pltpu.CompilerParams(
            dimension_semantics=("parallel",),
            vmem_limit_bytes=56 * 1024 * 1024,
        ),
        name="ntm_read_head",
        interpret=interpret,
    )(x, wp3, W, b2, mem)
    return r, w3


def kernel(x, w_prev, mem, W, b):
    r, w = _read_head(x, w_prev, mem, W, b)
    return (r, w)
